# baseline (speedup 1.0000x reference)
import jax
import jax.numpy as jnp
from jax.experimental import pallas as pl
from jax.experimental.pallas import tpu as pltpu

_N, _D, _E, _F = 2048, 1024, 8, 4096
_BT = 2048
_BF = 1024


def _moe_body(x_ref, rt_ref, w1_ref, b1_ref, w2_ref, o_ref):
    e = pl.program_id(1)
    f = pl.program_id(2)

    w1b = w1_ref[0].astype(jnp.bfloat16)
    h = jnp.dot(x_ref[...], w1b,
                preferred_element_type=jnp.float32).astype(jnp.bfloat16)
    h = h + b1_ref[0]
    h = h * (1.0 + jax.lax.erf(h * jnp.bfloat16(0.70710678)))
    h = h * rt_ref[0]
    y = jnp.dot(h, w2_ref[0].astype(jnp.bfloat16),
                preferred_element_type=jnp.float32)

    @pl.when(jnp.logical_and(e == 0, f == 0))
    def _init():
        o_ref[...] = y

    @pl.when(jnp.logical_or(e != 0, f != 0))
    def _acc():
        o_ref[...] += y


def kernel(x, routing_tensor, w1, b1, w2):
    x = x.astype(jnp.bfloat16)
    rt = (0.5 * routing_tensor.astype(jnp.bfloat16)).T.reshape(_E, _N, 1)
    b1r = b1.astype(jnp.bfloat16).reshape(_E, 1, _F)
    grid = (_N // _BT, _E, _F // _BF)
    return pl.pallas_call(
        _moe_body,
        grid=grid,
        in_specs=[
            pl.BlockSpec((_BT, _D), lambda t, e, f: (t, 0)),
            pl.BlockSpec((1, _BT, 1), lambda t, e, f: (e, t, 0)),
            pl.BlockSpec((1, _D, _BF), lambda t, e, f: (e, 0, f)),
            pl.BlockSpec((1, 1, _BF), lambda t, e, f: (e, 0, f)),
            pl.BlockSpec((1, _BF, _D), lambda t, e, f: (e, f, 0)),
        ],
        out_specs=pl.BlockSpec((_BT, _D), lambda t, e, f: (t, 0)),
        out_shape=jax.ShapeDtypeStruct((_N, _D), jnp.float32),
        compiler_params=pltpu.CompilerParams(
            dimension_semantics=("arbitrary", "arbitrary", "arbitrary"),
        ),
    )(x, rt, w1, b1r, w2)


# speedup vs baseline: 1.0710x; 1.0710x over previous
import jax
import jax.numpy as jnp
from jax.experimental import pallas as pl
from jax.experimental.pallas import tpu as pltpu

_N, _D, _E, _F = 2048, 1024, 8, 4096
_BT = 1024
_BF = 2048


def _moe_body(x_ref, rt_ref, w1_ref, b1_ref, w2_ref, o_ref):
    e = pl.program_id(1)
    f = pl.program_id(2)

    w1b = w1_ref[0].astype(jnp.bfloat16)
    h = jnp.dot(x_ref[...], w1b,
                preferred_element_type=jnp.float32).astype(jnp.bfloat16)
    h = h + b1_ref[0]
    h = h * (1.0 + jax.lax.erf(h * jnp.bfloat16(0.70710678)))
    h = h * rt_ref[0]
    y = jnp.dot(h, w2_ref[0].astype(jnp.bfloat16),
                preferred_element_type=jnp.float32)

    @pl.when(jnp.logical_and(e == 0, f == 0))
    def _init():
        o_ref[...] = y

    @pl.when(jnp.logical_or(e != 0, f != 0))
    def _acc():
        o_ref[...] += y


def kernel(x, routing_tensor, w1, b1, w2):
    x = x.astype(jnp.bfloat16)
    rt = (0.5 * routing_tensor.astype(jnp.bfloat16)).T.reshape(_E, _N, 1)
    b1r = b1.astype(jnp.bfloat16).reshape(_E, 1, _F)
    grid = (_N // _BT, _E, _F // _BF)
    return pl.pallas_call(
        _moe_body,
        grid=grid,
        in_specs=[
            pl.BlockSpec((_BT, _D), lambda t, e, f: (t, 0)),
            pl.BlockSpec((1, _BT, 1), lambda t, e, f: (e, t, 0)),
            pl.BlockSpec((1, _D, _BF), lambda t, e, f: (e, 0, f)),
            pl.BlockSpec((1, 1, _BF), lambda t, e, f: (e, 0, f)),
            pl.BlockSpec((1, _BF, _D), lambda t, e, f: (e, f, 0)),
        ],
        out_specs=pl.BlockSpec((_BT, _D), lambda t, e, f: (t, 0)),
        out_shape=jax.ShapeDtypeStruct((_N, _D), jnp.float32),
        compiler_params=pltpu.CompilerParams(
            dimension_semantics=("arbitrary", "arbitrary", "arbitrary"),
        ),
    )(x, rt, w1, b1r, w2)


# speedup vs baseline: 1.0805x; 1.0088x over previous
import jax
import jax.numpy as jnp
from jax.experimental import pallas as pl
from jax.experimental.pallas import tpu as pltpu

_N, _D, _E, _F = 2048, 1024, 8, 4096
_BT = 1024
_BF = 2048


def _moe_body(x_ref, r_ref, w1_ref, b1_ref, w2_ref, o_ref):
    e = pl.program_id(1)
    f = pl.program_id(2)

    lane = jax.lax.broadcasted_iota(jnp.int32, (_BT, _E), 1)
    rcol = jnp.sum(jnp.where(lane == e, r_ref[...], 0.0), axis=1,
                   keepdims=True)
    rcol = (0.5 * rcol).astype(jnp.bfloat16)

    w1b = w1_ref[0].astype(jnp.bfloat16)
    h = jnp.dot(x_ref[...], w1b,
                preferred_element_type=jnp.float32).astype(jnp.bfloat16)
    h = h + b1_ref[0]
    h = h * (1.0 + jax.lax.erf(h * jnp.bfloat16(0.70710678)))
    h = h * rcol
    y = jnp.dot(h, w2_ref[0].astype(jnp.bfloat16),
                preferred_element_type=jnp.float32)

    @pl.when(jnp.logical_and(e == 0, f == 0))
    def _init():
        o_ref[...] = y

    @pl.when(jnp.logical_or(e != 0, f != 0))
    def _acc():
        o_ref[...] += y


def kernel(x, routing_tensor, w1, b1, w2):
    x = x.astype(jnp.bfloat16)
    b1r = b1.astype(jnp.bfloat16).reshape(_E, 1, _F)
    grid = (_N // _BT, _E, _F // _BF)
    return pl.pallas_call(
        _moe_body,
        grid=grid,
        in_specs=[
            pl.BlockSpec((_BT, _D), lambda t, e, f: (t, 0)),
            pl.BlockSpec((_BT, _E), lambda t, e, f: (t, 0)),
            pl.BlockSpec((1, _D, _BF), lambda t, e, f: (e, 0, f)),
            pl.BlockSpec((1, 1, _BF), lambda t, e, f: (e, 0, f)),
            pl.BlockSpec((1, _BF, _D), lambda t, e, f: (e, f, 0)),
        ],
        out_specs=pl.BlockSpec((_BT, _D), lambda t, e, f: (t, 0)),
        out_shape=jax.ShapeDtypeStruct((_N, _D), jnp.float32),
        compiler_params=pltpu.CompilerParams(
            dimension_semantics=("arbitrary", "arbitrary", "arbitrary"),
            vmem_limit_bytes=64 * 1024 * 1024,
        ),
    )(x, routing_tensor, w1, b1r, w2)
